# baseline (speedup 1.0000x reference)
import functools

import jax
import jax.numpy as jnp
from jax import lax
from jax.experimental import pallas as pl
from jax.experimental.pallas import tpu as pltpu
from jax.experimental.pallas import tpu_sc as plsc

NUM_USERS = 100000
NUM_ITEMS = 100000
EMBED_DIM = 64
BATCH = 4096

_INFO = plsc.get_sparse_core_info()
_NC, _NS, _L = _INFO.num_cores, _INFO.num_subcores, _INFO.num_lanes
_NW = _NC * _NS
_BPW = BATCH // _NW
_GROUPS = _BPW // _L
_ROW = 2 * EMBED_DIM


def _sc_kernel(cat_hbm, uid_hbm, iid_hbm, out_hbm,
               uid_v, iid_v, gb_v, gc_v,
               cu1_v, cu2_v, ci1_v, ci2_v,
               rows_ua, rows_ub, rows_ia, rows_ib, out_v, sem):
    wid = lax.axis_index("s") * _NC + lax.axis_index("c")
    base = wid * _BPW

    pltpu.sync_copy(uid_hbm.at[pl.ds(base, _BPW)], uid_v)
    pltpu.sync_copy(iid_hbm.at[pl.ds(base, _BPW)], iid_v)

    for g in range(_GROUPS):
        sl = pl.ds(g * _L, _L)
        u = uid_v[sl]
        su = jnp.logical_and(jnp.equal(jnp.remainder(u, 1500), 0),
                             u <= 94500)
        gb_v[sl] = jnp.where(su, u + 3, u)
        half = jnp.full((_L,), 0.5, jnp.float32)
        quarter = jnp.full((_L,), 0.25, jnp.float32)
        zero = jnp.zeros((_L,), jnp.float32)
        cu1_v[sl] = jnp.where(su, half, quarter)
        cu2_v[sl] = jnp.where(su, half, zero)

        i = iid_v[sl]
        si = jnp.logical_and(
            jnp.logical_and(jnp.equal(jnp.remainder(i - 3, 1500), 0), i >= 3),
            i <= 94503)
        gc_v[sl] = jnp.where(si, i - 3, i)
        ci1_v[sl] = jnp.where(si, half, zero)
        ci2_v[sl] = jnp.where(si, half, quarter)

    c1 = pltpu.async_copy(cat_hbm.at[uid_v], rows_ua, sem)
    c2 = pltpu.async_copy(cat_hbm.at[gb_v], rows_ub, sem)
    c3 = pltpu.async_copy(cat_hbm.at[gc_v], rows_ia, sem)
    c4 = pltpu.async_copy(cat_hbm.at[iid_v], rows_ib, sem)
    c1.wait(); c2.wait(); c3.wait(); c4.wait()

    lane = lax.iota(jnp.int32, _L)
    for g in range(_GROUPS):
        sl = pl.ds(g * _L, _L)
        lrow = lane + g * _L
        cu1 = cu1_v[sl]
        cu2 = cu2_v[sl]
        ci1 = ci1_v[sl]
        ci2 = ci2_v[sl]

        def body(d, acc):
            col = jnp.bitwise_and(lane + d, EMBED_DIM - 1)
            coli = col + EMBED_DIM
            ua = plsc.load_gather(rows_ua, [lrow, col])
            ub = plsc.load_gather(rows_ub, [lrow, coli])
            ia = plsc.load_gather(rows_ia, [lrow, col])
            ib = plsc.load_gather(rows_ib, [lrow, coli])
            ue = cu1 * ua + cu2 * ub
            ie = ci1 * ia + ci2 * ib
            return acc + ue * ie

        out_v[sl] = lax.fori_loop(0, EMBED_DIM, body,
                                  jnp.zeros((_L,), jnp.float32))

    pltpu.sync_copy(out_v, out_hbm.at[pl.ds(base, _BPW)])


@jax.jit
def _run(user_emb, item_emb, user_ids, item_ids):
    mesh = plsc.VectorSubcoreMesh(core_axis_name="c", subcore_axis_name="s")
    kern = functools.partial(
        pl.kernel,
        mesh=mesh,
        compiler_params=pltpu.CompilerParams(
            needs_layout_passes=False, use_tc_tiling_on_sc=True),
        out_type=jax.ShapeDtypeStruct((BATCH,), jnp.float32),
        scratch_types=[
            pltpu.VMEM((_BPW,), jnp.int32),
            pltpu.VMEM((_BPW,), jnp.int32),
            pltpu.VMEM((_BPW,), jnp.int32),
            pltpu.VMEM((_BPW,), jnp.int32),
            pltpu.VMEM((_BPW,), jnp.float32),
            pltpu.VMEM((_BPW,), jnp.float32),
            pltpu.VMEM((_BPW,), jnp.float32),
            pltpu.VMEM((_BPW,), jnp.float32),
            pltpu.VMEM((_BPW, _ROW), jnp.float32),
            pltpu.VMEM((_BPW, _ROW), jnp.float32),
            pltpu.VMEM((_BPW, _ROW), jnp.float32),
            pltpu.VMEM((_BPW, _ROW), jnp.float32),
            pltpu.VMEM((_BPW,), jnp.float32),
            pltpu.SemaphoreType.DMA,
        ],
    )(_sc_kernel)
    cat = jnp.concatenate([user_emb, item_emb], axis=1)
    return kern(cat, user_ids, item_ids)


def kernel(user_emb, item_emb, user_ids, item_ids):
    return _run(user_emb, item_emb,
                user_ids.astype(jnp.int32), item_ids.astype(jnp.int32))
